# baseline (speedup 1.0000x reference)
import functools

import jax
import jax.numpy as jnp
from jax import lax
from jax.experimental import pallas as pl
from jax.experimental.pallas import tpu as pltpu
from jax.experimental.pallas import tpu_sc as plsc

_NC = 2
_NS = 16
_NW = _NC * _NS
_NEG = -3.0e38


def _g_body(f_ref, p_ref, w1t_ref, b1_ref, w2t_ref, b2_ref, g_ref):
    p = p_ref[...]
    h = (p[:, 0:1] * w1t_ref[0:1, :]
         + p[:, 1:2] * w1t_ref[1:2, :]
         + p[:, 2:3] * w1t_ref[2:3, :]
         + b1_ref[...])
    h = jnp.maximum(h, 0.0)
    bias = jnp.dot(h, w2t_ref[...], preferred_element_type=jnp.float32)
    g_ref[...] = (f_ref[...] + bias + b2_ref[...]).astype(jnp.bfloat16)


def _compute_g(f, p, W1, b1, W2, b2):
    n, c = f.shape
    br = 2000
    assert n % br == 0
    grid = n // br
    return pl.pallas_call(
        _g_body,
        grid=(grid,),
        in_specs=[
            pl.BlockSpec((br, c), lambda i: (i, 0)),
            pl.BlockSpec((br, 3), lambda i: (i, 0)),
            pl.BlockSpec((3, 32), lambda i: (0, 0)),
            pl.BlockSpec((1, 32), lambda i: (0, 0)),
            pl.BlockSpec((32, c), lambda i: (0, 0)),
            pl.BlockSpec((1, c), lambda i: (0, 0)),
        ],
        out_specs=pl.BlockSpec((br, c), lambda i: (i, 0)),
        out_shape=jax.ShapeDtypeStruct((n, c), jnp.bfloat16),
    )(f, p, W1.T, b1[None, :], W2.T, b2[None, :])


def _make_seg_max(N, C, M):
    ROWS = ((N + _NW - 1) // _NW + 7) // 8 * 8
    NPAD = ROWS * _NW
    SCAN = 8000
    assert M % (2 * SCAN) == 0 and SCAN % 16 == 0
    NCH = M // SCAN
    GB = 256
    CAPB = 16384
    RMASK = CAPB - 1
    ACC = ROWS * C
    UROWS = jnp.uint32(ROWS)

    mesh = plsc.VectorSubcoreMesh(
        core_axis_name="c", subcore_axis_name="s",
        num_cores=_NC, num_subcores=_NS)

    @functools.partial(
        pl.kernel,
        out_type=jax.ShapeDtypeStruct((NPAD * C,), jnp.bfloat16),
        mesh=mesh,
        compiler_params=pltpu.CompilerParams(
            needs_layout_passes=False, use_tc_tiling_on_sc=False),
        scratch_types=[
            pltpu.VMEM((ACC,), jnp.bfloat16),
            pltpu.VMEM((2 * SCAN,), jnp.int32),
            pltpu.VMEM((CAPB,), jnp.int32),
            pltpu.VMEM((CAPB + 16,), jnp.int32),
            pltpu.VMEM((2 * GB, C), jnp.bfloat16),
            pltpu.SemaphoreType.DMA,
            pltpu.SemaphoreType.DMA,
            pltpu.SemaphoreType.DMA,
        ],
    )
    def seg_max(g_hbm, edges_hbm, out_hbm,
                acc, estg, sbuf, dbuf, rows, sem0, sem1, gsem):
        w = lax.axis_index("s") * _NC + lax.axis_index("c")
        lo = w * ROWS

        def init_acc(i, _):
            acc[pl.ds(i * 32, 32)] = jnp.full((32,), _NEG, jnp.bfloat16)
            return 0
        lax.fori_loop(0, ACC // 32, init_acc, 0, unroll=4)

        def init_sbuf(i, _):
            sbuf[pl.ds(i * 16, 16)] = jnp.zeros((16,), jnp.int32)
            return 0
        lax.fori_loop(0, CAPB // 16, init_sbuf, 0, unroll=4)

        def stage_desc(c, half):
            base = half * SCAN
            sem = (sem0, sem1)[half]
            coff = pl.multiple_of(c * SCAN, 8)
            return pltpu.make_async_copy(
                edges_hbm.at[pl.ds(coff, SCAN)],
                estg.at[pl.ds(base, SCAN)], sem)

        def start_stage(c, half):
            stage_desc(c, half).start()

        def gather_desc(off, par):
            off = pl.multiple_of(off, 8)
            return pltpu.make_async_copy(
                g_hbm.at[sbuf.at[pl.ds(off, GB)]],
                rows.at[pl.ds(par * GB, GB), :], gsem)

        def merge_row(ab, re):
            a0 = acc[pl.ds(ab, 32)]
            r0 = rows[re, pl.ds(0, 32)]
            acc[pl.ds(ab, 32)] = jnp.maximum(a0, r0)
            a1 = acc[pl.ds(ab + 32, 32)]
            r1 = rows[re, pl.ds(32, 32)]
            acc[pl.ds(ab + 32, 32)] = jnp.maximum(a1, r1)

        def rmw_batch(off, count, par):
            if isinstance(count, int):
                assert count == GB

                def group(gi, _):
                    av = dbuf[pl.ds(off + gi * 16, 16)] * C
                    for k in range(16):
                        merge_row(av[k], par * GB + gi * 16 + k)
                    return 0
                lax.fori_loop(0, GB // 16, group, 0, unroll=2)
            else:
                def rmw(e, _):
                    dloc = dbuf[pl.ds(off + e, 16)][0]
                    merge_row(dloc * C, par * GB + e)
                    return 0
                lax.fori_loop(0, count, rmw, 0)

        def scan_chunk(half, pnd):
            base = half * SCAN

            def scan_body(i, pvm1):
                v = estg[pl.ds(base + i * 16, 16)]
                s = v & jnp.int32(0xFFFF)
                dl = lax.shift_right_logical(v, 16) - lo
                m = plsc.bitcast(dl, jnp.uint32) < UROWS
                cs = plsc.cumsum(jnp.where(m, 1, 0))
                pos = (pvm1 + cs) & RMASK
                plsc.store_scatter(sbuf, [pos], s, mask=m)
                plsc.store_scatter(dbuf, [pos], dl, mask=m)
                return pvm1 + plsc.all_reduce_population_count(m)
            return lax.fori_loop(0, SCAN // 16, scan_body, pnd, unroll=8)

        def drain(written, issued, done):
            def cond(state):
                iss, _ = state
                return iss + GB <= written

            def step(state):
                iss, dn = state

                @pl.when(iss > dn)
                def _():
                    gather_desc(dn & RMASK, (dn // GB) & 1).wait()
                    rmw_batch(dn & RMASK, GB, (dn // GB) & 1)
                dn = jnp.where(iss > dn, iss, dn)
                gather_desc(iss & RMASK, (iss // GB) & 1).start()
                return iss + GB, dn

            return lax.while_loop(cond, step, (issued, done))

        start_stage(0, 0)

        def pair_body(i, state):
            pnd, issued, done = state
            c0 = 2 * i

            start_stage(c0 + 1, 1)
            stage_desc(c0, 0).wait()
            pnd = scan_chunk(0, pnd)
            issued, done = drain(pnd[0] + 1, issued, done)

            @pl.when(c0 + 2 < NCH)
            def _():
                start_stage(c0 + 2, 0)
            stage_desc(c0 + 1, 1).wait()
            pnd = scan_chunk(1, pnd)
            issued, done = drain(pnd[0] + 1, issued, done)
            return pnd, issued, done

        neg1 = jnp.full((16,), -1, jnp.int32)
        pnd, issued, done = lax.fori_loop(
            0, NCH // 2, pair_body, (neg1, jnp.int32(0), jnp.int32(0)))

        @pl.when(issued > done)
        def _():
            gather_desc(done & RMASK, (done // GB) & 1).wait()
            rmw_batch(done & RMASK, GB, (done // GB) & 1)
        done = jnp.where(issued > done, issued, done)

        written = pnd[0] + 1
        gather_desc(done & RMASK, (done // GB) & 1).start()
        gather_desc(done & RMASK, (done // GB) & 1).wait()
        rmw_batch(done & RMASK, written - done, (done // GB) & 1)

        def writeback(i, _):
            v = acc[pl.ds(i * 32, 32)]
            acc[pl.ds(i * 32, 32)] = jnp.where(
                v == jnp.bfloat16(_NEG), jnp.bfloat16(0), v)
            return 0
        lax.fori_loop(0, ACC // 32, writeback, 0, unroll=4)
        pltpu.sync_copy(acc, out_hbm.at[pl.ds(pl.multiple_of(lo * C, 8), ACC)])

    return seg_max, NPAD


def kernel(f, p, rulebook, W1, b1, W2, b2):
    n, c = f.shape
    m = rulebook.shape[0]
    packed = rulebook[:, 0] | (rulebook[:, 1] << 16)
    g = _compute_g(f, p, W1, b1, W2, b2)
    seg_max, npad = _make_seg_max(n, c, m)
    out_flat = seg_max(g, packed)
    return out_flat.reshape(npad, c)[:n].astype(jnp.float32)


# speedup vs baseline: 1.9381x; 1.9381x over previous
import functools

import jax
import jax.numpy as jnp
from jax import lax
from jax.experimental import pallas as pl
from jax.experimental.pallas import tpu as pltpu
from jax.experimental.pallas import tpu_sc as plsc

_NC = 2
_NS = 16
_NW = _NC * _NS
_NEG = -3.0e38


def _g_body(f_ref, p_ref, w1t_ref, b1_ref, w2t_ref, b2_ref, g_ref):
    p = p_ref[...]
    h = (p[:, 0:1] * w1t_ref[0:1, :]
         + p[:, 1:2] * w1t_ref[1:2, :]
         + p[:, 2:3] * w1t_ref[2:3, :]
         + b1_ref[...])
    h = jnp.maximum(h, 0.0)
    bias = jnp.dot(h, w2t_ref[...], preferred_element_type=jnp.float32)
    g_ref[...] = (f_ref[...] + bias + b2_ref[...]).astype(jnp.bfloat16)


def _compute_g(f, p, W1, b1, W2, b2):
    n, c = f.shape
    br = 2000
    assert n % br == 0
    grid = n // br
    return pl.pallas_call(
        _g_body,
        grid=(grid,),
        in_specs=[
            pl.BlockSpec((br, c), lambda i: (i, 0)),
            pl.BlockSpec((br, 3), lambda i: (i, 0)),
            pl.BlockSpec((3, 32), lambda i: (0, 0)),
            pl.BlockSpec((1, 32), lambda i: (0, 0)),
            pl.BlockSpec((32, c), lambda i: (0, 0)),
            pl.BlockSpec((1, c), lambda i: (0, 0)),
        ],
        out_specs=pl.BlockSpec((br, c), lambda i: (i, 0)),
        out_shape=jax.ShapeDtypeStruct((n, c), jnp.bfloat16),
    )(f, p, W1.T, b1[None, :], W2.T, b2[None, :])


def _make_seg_max(N, C, M):
    ROWS = ((N + _NW - 1) // _NW + 7) // 8 * 8
    NPAD = ROWS * _NW
    SCAN = 8000
    assert M % (2 * SCAN) == 0 and SCAN % 16 == 0
    NCH = M // SCAN
    GB = 256
    CAPB = 16384
    RMASK = CAPB - 1
    ACC = ROWS * C
    UROWS = jnp.uint32(ROWS)

    mesh = plsc.VectorSubcoreMesh(
        core_axis_name="c", subcore_axis_name="s",
        num_cores=_NC, num_subcores=_NS)

    @functools.partial(
        pl.kernel,
        out_type=jax.ShapeDtypeStruct((NPAD * C,), jnp.bfloat16),
        mesh=mesh,
        compiler_params=pltpu.CompilerParams(
            needs_layout_passes=False, use_tc_tiling_on_sc=False),
        scratch_types=[
            pltpu.VMEM((ACC,), jnp.bfloat16),
            pltpu.VMEM((2 * SCAN,), jnp.int32),
            pltpu.VMEM((CAPB,), jnp.int32),
            pltpu.VMEM((CAPB + 16,), jnp.int32),
            pltpu.VMEM((2 * GB, C), jnp.bfloat16),
            pltpu.SemaphoreType.DMA,
            pltpu.SemaphoreType.DMA,
            pltpu.SemaphoreType.DMA,
        ],
    )
    def seg_max(g_hbm, edges_hbm, out_hbm,
                acc, estg, sbuf, dbuf, rows, sem0, sem1, gsem):
        w = lax.axis_index("s") * _NC + lax.axis_index("c")
        lo = w * ROWS

        def init_acc(i, _):
            acc[pl.ds(i * 32, 32)] = jnp.full((32,), _NEG, jnp.bfloat16)
            return 0
        lax.fori_loop(0, ACC // 32, init_acc, 0, unroll=4)

        def init_sbuf(i, _):
            sbuf[pl.ds(i * 16, 16)] = jnp.zeros((16,), jnp.int32)
            return 0
        lax.fori_loop(0, CAPB // 16, init_sbuf, 0, unroll=4)

        def stage_desc(c, half):
            base = half * SCAN
            sem = (sem0, sem1)[half]
            coff = pl.multiple_of(c * SCAN, 8)
            return pltpu.make_async_copy(
                edges_hbm.at[pl.ds(coff, SCAN)],
                estg.at[pl.ds(base, SCAN)], sem)

        def start_stage(c, half):
            stage_desc(c, half).start()

        def gather_desc(off, par):
            off = pl.multiple_of(off, 8)
            return pltpu.make_async_copy(
                g_hbm.at[sbuf.at[pl.ds(off, GB)]],
                rows.at[pl.ds(par * GB, GB), :], gsem)

        def merge_row(ab, re):
            a0 = acc[pl.ds(ab, 32)]
            r0 = rows[re, pl.ds(0, 32)]
            acc[pl.ds(ab, 32)] = jnp.maximum(a0, r0)
            a1 = acc[pl.ds(ab + 32, 32)]
            r1 = rows[re, pl.ds(32, 32)]
            acc[pl.ds(ab + 32, 32)] = jnp.maximum(a1, r1)

        def rmw_batch(off, count, par):
            if isinstance(count, int):
                assert count == GB

                def group(gi, _):
                    av = dbuf[pl.ds(off + gi * 16, 16)] * C
                    for k in range(16):
                        merge_row(av[k], par * GB + gi * 16 + k)
                    return 0
                lax.fori_loop(0, GB // 16, group, 0, unroll=2)
            else:
                def rmw(e, _):
                    dloc = dbuf[pl.ds(off + e, 16)][0]
                    merge_row(dloc * C, par * GB + e)
                    return 0
                lax.fori_loop(0, count, rmw, 0)

        def scan_chunk(half, pnd):
            base = half * SCAN

            def scan_body(i, pvm1):
                v = estg[pl.ds(base + i * 16, 16)]
                s = v & jnp.int32(0xFFFF)
                dl = lax.shift_right_logical(v, 16) - lo
                m = plsc.bitcast(dl, jnp.uint32) < UROWS
                cs = plsc.cumsum(jnp.where(m, 1, 0))
                pos = (pvm1 + cs) & RMASK
                plsc.store_scatter(sbuf, [pos], s, mask=m)
                plsc.store_scatter(dbuf, [pos], dl, mask=m)
                return pvm1 + plsc.all_reduce_population_count(m)
            return plsc.parallel_loop(
                0, SCAN // 16, carry=pnd, unroll=8)(scan_body)

        def drain(written, issued, done):
            def cond(state):
                iss, _ = state
                return iss + GB <= written

            def step(state):
                iss, dn = state

                @pl.when(iss > dn)
                def _():
                    gather_desc(dn & RMASK, (dn // GB) & 1).wait()
                    rmw_batch(dn & RMASK, GB, (dn // GB) & 1)
                dn = jnp.where(iss > dn, iss, dn)
                gather_desc(iss & RMASK, (iss // GB) & 1).start()
                return iss + GB, dn

            return lax.while_loop(cond, step, (issued, done))

        start_stage(0, 0)

        def pair_body(i, state):
            pnd, issued, done = state
            c0 = 2 * i

            start_stage(c0 + 1, 1)
            stage_desc(c0, 0).wait()
            pnd = scan_chunk(0, pnd)
            issued, done = drain(pnd[0] + 1, issued, done)

            @pl.when(c0 + 2 < NCH)
            def _():
                start_stage(c0 + 2, 0)
            stage_desc(c0 + 1, 1).wait()
            pnd = scan_chunk(1, pnd)
            issued, done = drain(pnd[0] + 1, issued, done)
            return pnd, issued, done

        neg1 = jnp.full((16,), -1, jnp.int32)
        pnd, issued, done = lax.fori_loop(
            0, NCH // 2, pair_body, (neg1, jnp.int32(0), jnp.int32(0)))

        @pl.when(issued > done)
        def _():
            gather_desc(done & RMASK, (done // GB) & 1).wait()
            rmw_batch(done & RMASK, GB, (done // GB) & 1)
        done = jnp.where(issued > done, issued, done)

        written = pnd[0] + 1
        gather_desc(done & RMASK, (done // GB) & 1).start()
        gather_desc(done & RMASK, (done // GB) & 1).wait()
        rmw_batch(done & RMASK, written - done, (done // GB) & 1)

        def writeback(i, _):
            v = acc[pl.ds(i * 32, 32)]
            acc[pl.ds(i * 32, 32)] = jnp.where(
                v == jnp.bfloat16(_NEG), jnp.bfloat16(0), v)
            return 0
        lax.fori_loop(0, ACC // 32, writeback, 0, unroll=4)
        pltpu.sync_copy(acc, out_hbm.at[pl.ds(pl.multiple_of(lo * C, 8), ACC)])

    return seg_max, NPAD


def kernel(f, p, rulebook, W1, b1, W2, b2):
    n, c = f.shape
    m = rulebook.shape[0]
    packed = rulebook[:, 0] | (rulebook[:, 1] << 16)
    g = _compute_g(f, p, W1, b1, W2, b2)
    seg_max, npad = _make_seg_max(n, c, m)
    out_flat = seg_max(g, packed)
    return out_flat.reshape(npad, c)[:n].astype(jnp.float32)


# speedup vs baseline: 1.9480x; 1.0051x over previous
import functools

import jax
import jax.numpy as jnp
from jax import lax
from jax.experimental import pallas as pl
from jax.experimental.pallas import tpu as pltpu
from jax.experimental.pallas import tpu_sc as plsc

_NC = 2
_NS = 16
_NW = _NC * _NS
_NEG = -3.0e38


def _g_body(f_ref, p_ref, w1t_ref, b1_ref, w2t_ref, b2_ref, g_ref):
    p = p_ref[...]
    h = (p[:, 0:1] * w1t_ref[0:1, :]
         + p[:, 1:2] * w1t_ref[1:2, :]
         + p[:, 2:3] * w1t_ref[2:3, :]
         + b1_ref[...])
    h = jnp.maximum(h, 0.0)
    bias = jnp.dot(h, w2t_ref[...], preferred_element_type=jnp.float32)
    g_ref[...] = (f_ref[...] + bias + b2_ref[...]).astype(jnp.bfloat16)


def _compute_g(f, p, W1, b1, W2, b2):
    n, c = f.shape
    br = 2000
    assert n % br == 0
    grid = n // br
    return pl.pallas_call(
        _g_body,
        grid=(grid,),
        in_specs=[
            pl.BlockSpec((br, c), lambda i: (i, 0)),
            pl.BlockSpec((br, 3), lambda i: (i, 0)),
            pl.BlockSpec((3, 32), lambda i: (0, 0)),
            pl.BlockSpec((1, 32), lambda i: (0, 0)),
            pl.BlockSpec((32, c), lambda i: (0, 0)),
            pl.BlockSpec((1, c), lambda i: (0, 0)),
        ],
        out_specs=pl.BlockSpec((br, c), lambda i: (i, 0)),
        out_shape=jax.ShapeDtypeStruct((n, c), jnp.bfloat16),
    )(f, p, W1.T, b1[None, :], W2.T, b2[None, :])


def _make_seg_max(N, C, M):
    ROWS = ((N + _NW - 1) // _NW + 7) // 8 * 8
    NPAD = ROWS * _NW
    SCAN = 8000
    assert M % (2 * SCAN) == 0 and SCAN % 16 == 0
    NCH = M // SCAN
    GB = 256
    CAPB = 16384
    RMASK = CAPB - 1
    ACC = ROWS * C
    UROWS = jnp.uint32(ROWS)

    mesh = plsc.VectorSubcoreMesh(
        core_axis_name="c", subcore_axis_name="s",
        num_cores=_NC, num_subcores=_NS)

    @functools.partial(
        pl.kernel,
        out_type=jax.ShapeDtypeStruct((NPAD * C,), jnp.bfloat16),
        mesh=mesh,
        compiler_params=pltpu.CompilerParams(
            needs_layout_passes=False, use_tc_tiling_on_sc=False),
        scratch_types=[
            pltpu.VMEM((ACC,), jnp.bfloat16),
            pltpu.VMEM((2 * SCAN,), jnp.int32),
            pltpu.VMEM((CAPB,), jnp.int32),
            pltpu.VMEM((CAPB + 16,), jnp.int32),
            pltpu.VMEM((2 * GB, C), jnp.bfloat16),
            pltpu.SemaphoreType.DMA,
            pltpu.SemaphoreType.DMA,
            pltpu.SemaphoreType.DMA,
        ],
    )
    def seg_max(g_hbm, edges_hbm, out_hbm,
                acc, estg, sbuf, dbuf, rows, sem0, sem1, gsem):
        w = lax.axis_index("s") * _NC + lax.axis_index("c")
        lo = w * ROWS

        @plsc.parallel_loop(0, ACC // 32, unroll=8)
        def _(i):
            acc[pl.ds(i * 32, 32)] = jnp.full((32,), _NEG, jnp.bfloat16)

        @plsc.parallel_loop(0, CAPB // 16, unroll=8)
        def _(i):
            sbuf[pl.ds(i * 16, 16)] = jnp.zeros((16,), jnp.int32)

        def stage_desc(c, half):
            base = half * SCAN
            sem = (sem0, sem1)[half]
            coff = pl.multiple_of(c * SCAN, 8)
            return pltpu.make_async_copy(
                edges_hbm.at[pl.ds(coff, SCAN)],
                estg.at[pl.ds(base, SCAN)], sem)

        def start_stage(c, half):
            stage_desc(c, half).start()

        def gather_desc(off, par):
            off = pl.multiple_of(off, 8)
            return pltpu.make_async_copy(
                g_hbm.at[sbuf.at[pl.ds(off, GB)]],
                rows.at[pl.ds(par * GB, GB), :], gsem)

        def merge_row(ab, re):
            a0 = acc[pl.ds(ab, 32)]
            r0 = rows[re, pl.ds(0, 32)]
            acc[pl.ds(ab, 32)] = jnp.maximum(a0, r0)
            a1 = acc[pl.ds(ab + 32, 32)]
            r1 = rows[re, pl.ds(32, 32)]
            acc[pl.ds(ab + 32, 32)] = jnp.maximum(a1, r1)

        def rmw_batch(off, count, par):
            if isinstance(count, int):
                assert count == GB

                def group(gi, _):
                    av = dbuf[pl.ds(off + gi * 16, 16)] * C
                    for k in range(16):
                        merge_row(av[k], par * GB + gi * 16 + k)
                    return 0
                lax.fori_loop(0, GB // 16, group, 0, unroll=2)
            else:
                def rmw(e, _):
                    dloc = dbuf[pl.ds(off + e, 16)][0]
                    merge_row(dloc * C, par * GB + e)
                    return 0
                lax.fori_loop(0, count, rmw, 0)

        def scan_chunk(half, pnd):
            base = half * SCAN

            def scan_body(i, pvm1):
                v = estg[pl.ds(base + i * 16, 16)]
                s = v & jnp.int32(0xFFFF)
                dl = lax.shift_right_logical(v, 16) - lo
                m = plsc.bitcast(dl, jnp.uint32) < UROWS
                cs = plsc.cumsum(jnp.where(m, 1, 0))
                pos = (pvm1 + cs) & RMASK
                plsc.store_scatter(sbuf, [pos], s, mask=m)
                plsc.store_scatter(dbuf, [pos], dl, mask=m)
                return pvm1 + plsc.all_reduce_population_count(m)
            return plsc.parallel_loop(
                0, SCAN // 16, carry=pnd, unroll=16)(scan_body)

        def drain(written, issued, done):
            def cond(state):
                iss, _ = state
                return iss + GB <= written

            def step(state):
                iss, dn = state

                @pl.when(iss > dn)
                def _():
                    gather_desc(dn & RMASK, (dn // GB) & 1).wait()
                    rmw_batch(dn & RMASK, GB, (dn // GB) & 1)
                dn = jnp.where(iss > dn, iss, dn)
                gather_desc(iss & RMASK, (iss // GB) & 1).start()
                return iss + GB, dn

            return lax.while_loop(cond, step, (issued, done))

        start_stage(0, 0)

        def pair_body(i, state):
            pnd, issued, done = state
            c0 = 2 * i

            start_stage(c0 + 1, 1)
            stage_desc(c0, 0).wait()
            pnd = scan_chunk(0, pnd)
            issued, done = drain(pnd[0] + 1, issued, done)

            @pl.when(c0 + 2 < NCH)
            def _():
                start_stage(c0 + 2, 0)
            stage_desc(c0 + 1, 1).wait()
            pnd = scan_chunk(1, pnd)
            issued, done = drain(pnd[0] + 1, issued, done)
            return pnd, issued, done

        neg1 = jnp.full((16,), -1, jnp.int32)
        pnd, issued, done = lax.fori_loop(
            0, NCH // 2, pair_body, (neg1, jnp.int32(0), jnp.int32(0)))

        @pl.when(issued > done)
        def _():
            gather_desc(done & RMASK, (done // GB) & 1).wait()
            rmw_batch(done & RMASK, GB, (done // GB) & 1)
        done = jnp.where(issued > done, issued, done)

        written = pnd[0] + 1
        gather_desc(done & RMASK, (done // GB) & 1).start()
        gather_desc(done & RMASK, (done // GB) & 1).wait()
        rmw_batch(done & RMASK, written - done, (done // GB) & 1)

        @plsc.parallel_loop(0, ACC // 32, unroll=8)
        def _(i):
            v = acc[pl.ds(i * 32, 32)]
            acc[pl.ds(i * 32, 32)] = jnp.where(
                v == jnp.bfloat16(_NEG), jnp.bfloat16(0), v)
        pltpu.sync_copy(acc, out_hbm.at[pl.ds(pl.multiple_of(lo * C, 8), ACC)])

    return seg_max, NPAD


def kernel(f, p, rulebook, W1, b1, W2, b2):
    n, c = f.shape
    m = rulebook.shape[0]
    packed = rulebook[:, 0] | (rulebook[:, 1] << 16)
    g = _compute_g(f, p, W1, b1, W2, b2)
    seg_max, npad = _make_seg_max(n, c, m)
    out_flat = seg_max(g, packed)
    return out_flat.reshape(npad, c)[:n].astype(jnp.float32)
